# Initial kernel scaffold; baseline (speedup 1.0000x reference)
import jax
import jax.numpy as jnp
from jax.experimental import pallas as pl


def kernel(seq, token_table, seg_table, gamma, beta):
    raise NotImplementedError("write your pallas kernel here")


# speedup vs baseline: 2.3236x; 2.3236x over previous
import functools

import jax
import jax.numpy as jnp
import numpy as np
from jax import lax
from jax.experimental import pallas as pl
from jax.experimental.pallas import tpu as pltpu
from jax.experimental.pallas import tpu_sc as plsc

_VOCAB = 100000
_EMB = 64
_BATCH = 1024
_SEQLEN = 200

_NC = 2
_NS = 16
_NW = _NC * _NS
_TOKENS = _BATCH * _SEQLEN
_PER_W = _TOKENS // _NW
_CH = 40
_NCH = _PER_W // _CH


def _pos_plus_seg(seg_table):
    pos = jnp.arange(_SEQLEN, dtype=jnp.float32)[:, None]
    d = jnp.arange(_EMB, dtype=jnp.float32)
    d = 2.0 * d / _EMB
    p = pos / jnp.power(10000.0, d)
    p = p.at[:, ::2].set(jnp.sin(p[:, ::2]))
    p = p.at[:, 1::2].set(jnp.cos(p[:, 1::2]))
    seg_sel = (jnp.arange(_SEQLEN) >= _SEQLEN // 2 + 1)[:, None]
    seg = jnp.where(seg_sel, seg_table[1][None, :], seg_table[0][None, :])
    return p + seg


_GATHER_DNUMS = lax.GatherDimensionNumbers(
    offset_dims=(), collapsed_slice_dims=(0,), start_index_map=(0,))
def _bfly_perms():
    lane = lax.iota(jnp.int32, 16)
    return [(lane ^ k).reshape(16, 1) for k in (1, 2, 4, 8)]


def _lane_sum(x, perms):
    for idx in perms:
        x = x + lax.gather(x, idx, dimension_numbers=_GATHER_DNUMS,
                           slice_sizes=(1,),
                           mode=lax.GatherScatterMode.PROMISE_IN_BOUNDS)
    return x


def _sc_body(idx_hbm, table_hbm, cmb_hbm, g_hbm, b_hbm, out_hbm,
             idx_v, rows_v, cmb_v, g_v, b_v, sem):
    wid = lax.axis_index("s") * _NC + lax.axis_index("c")
    pltpu.sync_copy(cmb_hbm, cmb_v)
    pltpu.sync_copy(g_hbm, g_v)
    pltpu.sync_copy(b_hbm, b_v)
    base = wid * _PER_W

    g_regs = [g_v[pl.ds(16 * j, 16)] for j in range(4)]
    b_regs = [b_v[pl.ds(16 * j, 16)] for j in range(4)]
    perms = _bfly_perms()

    def chunk_body(c, carry):
        flat = base + c * _CH
        pltpu.sync_copy(idx_hbm.at[pl.ds(flat, _CH)], idx_v)
        pltpu.async_copy(table_hbm.at[idx_v], rows_v, sem).wait()
        pbase = lax.rem(c * _CH, _SEQLEN)

        def row_body(r, rcarry):
            p = pbase + r
            v = [rows_v[r, pl.ds(16 * j, 16)] + cmb_v[p, pl.ds(16 * j, 16)]
                 for j in range(4)]
            s = (v[0] + v[1]) + (v[2] + v[3])
            q = ((v[0] * v[0] + v[1] * v[1]) + (v[2] * v[2] + v[3] * v[3]))
            mean = _lane_sum(s, perms) * (1.0 / 64.0)
            varv = _lane_sum(q, perms) * (1.0 / 64.0) - mean * mean + 1e-5
            iv = lax.bitcast_convert_type(varv, jnp.int32)
            y = lax.bitcast_convert_type(
                jnp.int32(0x5F3759DF) - lax.shift_right_logical(iv, 1),
                jnp.float32)
            xh = varv * 0.5
            for _ in range(3):
                y = y * (1.5 - xh * y * y)
            for j in range(4):
                rows_v[r, pl.ds(16 * j, 16)] = (
                    (v[j] - mean) * y * g_regs[j] + b_regs[j])
            return rcarry

        lax.fori_loop(0, _CH, row_body, 0)
        pltpu.sync_copy(rows_v, out_hbm.at[pl.ds(flat, _CH)])
        return carry

    lax.fori_loop(0, _NCH, chunk_body, 0)


def kernel(seq, token_table, seg_table, gamma, beta):
    cmb = _pos_plus_seg(seg_table)
    idx = seq.reshape(_TOKENS)
    run = functools.partial(
        pl.kernel,
        out_type=jax.ShapeDtypeStruct((_TOKENS, _EMB), jnp.float32),
        mesh=plsc.VectorSubcoreMesh(core_axis_name="c", subcore_axis_name="s"),
        scratch_types=[
            pltpu.VMEM((_CH,), jnp.int32),
            pltpu.VMEM((_CH, _EMB), jnp.float32),
            pltpu.VMEM((_SEQLEN, _EMB), jnp.float32),
            pltpu.VMEM((_EMB,), jnp.float32),
            pltpu.VMEM((_EMB,), jnp.float32),
            pltpu.SemaphoreType.DMA,
        ],
        compiler_params=pltpu.CompilerParams(use_tc_tiling_on_sc=False),
    )(_sc_body)
    out = run(idx, token_table, cmb, gamma, beta)
    return out.reshape(_BATCH, _SEQLEN, _EMB)


# speedup vs baseline: 5.6584x; 2.4352x over previous
import functools

import jax
import jax.numpy as jnp
import numpy as np
from jax import lax
from jax.experimental import pallas as pl
from jax.experimental.pallas import tpu as pltpu
from jax.experimental.pallas import tpu_sc as plsc

_VOCAB = 100000
_EMB = 64
_BATCH = 1024
_SEQLEN = 200

_NC = 2
_NS = 16
_NW = _NC * _NS
_TOKENS = _BATCH * _SEQLEN
_PER_W = _TOKENS // _NW
_CH = 128
_NCH = _PER_W // _CH


def _pos_plus_seg(seg_table):
    pos = jnp.arange(_SEQLEN, dtype=jnp.float32)[:, None]
    d = jnp.arange(_EMB, dtype=jnp.float32)
    d = 2.0 * d / _EMB
    p = pos / jnp.power(10000.0, d)
    p = p.at[:, ::2].set(jnp.sin(p[:, ::2]))
    p = p.at[:, 1::2].set(jnp.cos(p[:, 1::2]))
    seg_sel = (jnp.arange(_SEQLEN) >= _SEQLEN // 2 + 1)[:, None]
    seg = jnp.where(seg_sel, seg_table[1][None, :], seg_table[0][None, :])
    return p + seg


_GATHER_DNUMS = lax.GatherDimensionNumbers(
    offset_dims=(), collapsed_slice_dims=(0,), start_index_map=(0,))
def _bfly_perms():
    lane = lax.iota(jnp.int32, 16)
    return [(lane ^ k).reshape(16, 1) for k in (1, 2, 4, 8)]


def _lane_sum(x, perms):
    for idx in perms:
        x = x + lax.gather(x, idx, dimension_numbers=_GATHER_DNUMS,
                           slice_sizes=(1,),
                           mode=lax.GatherScatterMode.PROMISE_IN_BOUNDS)
    return x


def _sc_body(idx_hbm, table_hbm, cmb_hbm, g_hbm, b_hbm, out_hbm,
             idx_v, rows_v, cmb_v, g_v, b_v, sem):
    wid = lax.axis_index("s") * _NC + lax.axis_index("c")
    pltpu.sync_copy(cmb_hbm, cmb_v)
    pltpu.sync_copy(g_hbm, g_v)
    pltpu.sync_copy(b_hbm, b_v)
    base = wid * _PER_W

    g_regs = [g_v[pl.ds(16 * j, 16)] for j in range(4)]
    b_regs = [b_v[pl.ds(16 * j, 16)] for j in range(4)]
    perms = _bfly_perms()
    pltpu.sync_copy(idx_hbm.at[pl.ds(base, _PER_W)], idx_v)

    def chunk_body(c, carry):
        flat = base + c * _CH
        cbase = c * _CH
        pltpu.async_copy(
            table_hbm.at[idx_v.at[pl.ds(cbase, _CH)]], rows_v, sem).wait()

        @functools.partial(plsc.parallel_loop, 0, _CH, unroll=4)
        def row_body(r):
            p = lax.rem(cbase + r, _SEQLEN)
            v = [rows_v[r, pl.ds(16 * j, 16)] + cmb_v[p, pl.ds(16 * j, 16)]
                 for j in range(4)]
            s = (v[0] + v[1]) + (v[2] + v[3])
            q = ((v[0] * v[0] + v[1] * v[1]) + (v[2] * v[2] + v[3] * v[3]))
            mean = _lane_sum(s, perms) * (1.0 / 64.0)
            varv = _lane_sum(q, perms) * (1.0 / 64.0) - mean * mean + 1e-5
            iv = lax.bitcast_convert_type(varv, jnp.int32)
            y = lax.bitcast_convert_type(
                jnp.int32(0x5F375A86) - lax.shift_right_logical(iv, 1),
                jnp.float32)
            xh = varv * 0.5
            for _ in range(2):
                y = y * (1.5 - xh * y * y)
            for j in range(4):
                rows_v[r, pl.ds(16 * j, 16)] = (
                    (v[j] - mean) * y * g_regs[j] + b_regs[j])

        pltpu.sync_copy(rows_v, out_hbm.at[pl.ds(flat, _CH)])
        return carry

    lax.fori_loop(0, _NCH, chunk_body, 0)


def kernel(seq, token_table, seg_table, gamma, beta):
    cmb = _pos_plus_seg(seg_table)
    idx = seq.reshape(_TOKENS)
    run = functools.partial(
        pl.kernel,
        out_type=jax.ShapeDtypeStruct((_TOKENS, _EMB), jnp.float32),
        mesh=plsc.VectorSubcoreMesh(core_axis_name="c", subcore_axis_name="s"),
        scratch_types=[
            pltpu.VMEM((_PER_W,), jnp.int32),
            pltpu.VMEM((_CH, _EMB), jnp.float32),
            pltpu.VMEM((_SEQLEN, _EMB), jnp.float32),
            pltpu.VMEM((_EMB,), jnp.float32),
            pltpu.VMEM((_EMB,), jnp.float32),
            pltpu.SemaphoreType.DMA,
        ],
        compiler_params=pltpu.CompilerParams(use_tc_tiling_on_sc=False),
    )(_sc_body)
    out = run(idx, token_table, cmb, gamma, beta)
    return out.reshape(_BATCH, _SEQLEN, _EMB)
